# baseline (speedup 1.0000x reference)
import jax
import jax.numpy as jnp
from jax.experimental import pallas as pl
from jax.experimental.pallas import tpu as pltpu

N = 4096
EMB = 128
KPTS = 64
NB = 8
DP = 32
BN = 512
NBLK = N // BN
_INV_SQRT2 = 0.7071067811865475
_SILU_SCALE = 1.0 / 0.6


def _scaled_silu(v):
    return jax.nn.sigmoid(v) * v * _SILU_SCALE


def _dense_t(v, w):
    out = jax.lax.dot_general(v, w, (((1,), (1,)), ((), ())),
                              preferred_element_type=jnp.float32)
    return _scaled_silu(out)


def _residual(v, w1, w2):
    return (v + _dense_t(_dense_t(v, w1), w2)) * _INV_SQRT2


def _fused(h_ref, xt_ref, seg_ref, kt_ref, w1_ref, w2_ref, dw_ref, uw_ref,
           ew_ref, r1a_ref, r1b_ref, r2a_ref, r2b_ref,
           dot_ref, out_ref,
           cos_s, sin_s, dot_s, sfr_s, sfi_s):
    p = pl.program_id(0)
    j = pl.program_id(1)
    rows = pl.ds(j * BN, BN)

    @pl.when(p == 0)
    def _phase_a():
        seg = seg_ref[...]
        hres = _residual(h_ref[...], w1_ref[...], w2_ref[...])

        lane24 = jax.lax.broadcasted_iota(jnp.int32, (BN, NB * 3), 1)
        xe = jnp.where(seg == lane24 // 3, xt_ref[...], 0.0)
        dot = jax.lax.dot_general(xe, kt_ref[...], (((1,), (0,)), ((), ())),
                                  preferred_element_type=jnp.float32,
                                  precision=jax.lax.Precision.HIGHEST)
        cosd = jnp.cos(dot)
        sind = jnp.sin(dot)
        dot_s[rows, :] = dot

        masks = [jnp.where(seg == b, 1.0, 0.0) for b in range(NB)]
        tr = jnp.concatenate([cosd * m for m in masks], axis=1)
        ti = jnp.concatenate([sind * m for m in masks], axis=1)
        cos_s[rows, :] = cosd
        sin_s[rows, :] = sind

        dn = (((0,), (0,)), ((), ()))
        sr = jax.lax.dot_general(tr, hres, dn,
                                 preferred_element_type=jnp.float32)
        si = jax.lax.dot_general(ti, hres, dn,
                                 preferred_element_type=jnp.float32)

        @pl.when(j == 0)
        def _init():
            sfr_s[...] = sr
            sfi_s[...] = si

        @pl.when(j > 0)
        def _acc():
            sfr_s[...] += sr
            sfi_s[...] += si

    @pl.when(p == 1)
    def _phase_b():
        kf = jax.lax.dot_general(dw_ref[...], uw_ref[...],
                                 (((0,), (1,)), ((), ())),
                                 preferred_element_type=jnp.float32)
        ktile = jnp.concatenate([kf] * NB, axis=0)
        ar = sfr_s[...] * ktile
        ai = sfi_s[...] * ktile

        seg = seg_ref[...]
        cosd = cos_s[rows, :]
        sind = sin_s[rows, :]
        masks = [jnp.where(seg == b, 1.0, 0.0) for b in range(NB)]
        tr = jnp.concatenate([cosd * m for m in masks], axis=1)
        ti = jnp.concatenate([sind * m for m in masks], axis=1)
        hu = 0.01 * (jnp.dot(tr, ar, preferred_element_type=jnp.float32)
                     + jnp.dot(ti, ai, preferred_element_type=jnp.float32))
        hu = _dense_t(hu, ew_ref[...])
        hu = _residual(hu, r1a_ref[...], r1b_ref[...])
        hu = _residual(hu, r2a_ref[...], r2b_ref[...])
        out_ref[...] = hu
        dot_ref[...] = dot_s[rows, :]


@jax.jit
def _run(h, xt, seg_col, kt, down_w, up_w, pre_w1, pre_w2, ew_w,
         r1w1, r1w2, r2w1, r2w2):
    ph_a = lambda p, j: ((1 - p) * j, 0)
    ph_b = lambda p, j: (p * j, 0)
    rep = lambda p, j: (0, 0)

    dot, h_update = pl.pallas_call(
        _fused,
        grid=(2, NBLK),
        in_specs=[
            pl.BlockSpec((BN, EMB), ph_a),
            pl.BlockSpec((BN, NB * 3), ph_a),
            pl.BlockSpec((BN, 1), lambda p, j: (j, 0)),
            pl.BlockSpec((NB * 3, KPTS), rep),
            pl.BlockSpec((EMB, EMB), rep),
            pl.BlockSpec((EMB, EMB), rep),
            pl.BlockSpec((DP, KPTS), rep),
            pl.BlockSpec((EMB, DP), rep),
            pl.BlockSpec((EMB, EMB), rep),
            pl.BlockSpec((EMB, EMB), rep),
            pl.BlockSpec((EMB, EMB), rep),
            pl.BlockSpec((EMB, EMB), rep),
            pl.BlockSpec((EMB, EMB), rep),
        ],
        out_specs=[
            pl.BlockSpec((BN, KPTS), ph_b),
            pl.BlockSpec((BN, EMB), ph_b),
        ],
        out_shape=[
            jax.ShapeDtypeStruct((N, KPTS), jnp.float32),
            jax.ShapeDtypeStruct((N, EMB), jnp.float32),
        ],
        scratch_shapes=[
            pltpu.VMEM((N, KPTS), jnp.float32),
            pltpu.VMEM((N, KPTS), jnp.float32),
            pltpu.VMEM((N, KPTS), jnp.float32),
            pltpu.VMEM((NB * KPTS, EMB), jnp.float32),
            pltpu.VMEM((NB * KPTS, EMB), jnp.float32),
        ],
    )(h, xt, seg_col, kt, pre_w1, pre_w2, down_w, up_w, ew_w,
      r1w1, r1w2, r2w1, r2w2)

    return h_update, dot


def kernel(h, x, k, num_batch, batch_seg, down_w, up_w, pre_w1, pre_w2,
           ew_w, r1w1, r1w2, r2w1, r2w2):
    kt = jnp.transpose(k, (0, 2, 1)).reshape(NB * 3, KPTS)
    xt = jnp.tile(x, (1, NB))
    seg_col = batch_seg.reshape(N, 1).astype(jnp.int32)
    h_update, dot = _run(h, xt, seg_col, kt, down_w, up_w, pre_w1, pre_w2,
                         ew_w, r1w1, r1w2, r2w1, r2w2)
    return h_update, dot, jnp.asarray(1.0, dtype=jnp.float32)
